# baseline (speedup 1.0000x reference)
import functools

import jax
import jax.numpy as jnp
from jax import lax
from jax.experimental import pallas as pl
from jax.experimental.pallas import tpu as pltpu
from jax.experimental.pallas import tpu_sc as plsc

EMB = 64
HID = 128
EPS = 1e-12

NC = 2
NS = 16
NW = NC * NS

CHUNK = 800

TBLK2 = 4000


def _dense_pairs_body(t_ref, w2_ref, b2_ref, g_ref, beta_ref, o_ref):
    x2 = t_ref[...]
    h2 = jnp.dot(x2, w2_ref[...], preferred_element_type=jnp.float32)
    h2 = h2 + b2_ref[...]
    he = h2[:, :HID]
    ho = h2[:, HID:]

    def ln(hc):
        var = jnp.mean(hc * hc, axis=-1, keepdims=True)
        return hc * lax.rsqrt(var + EPS) * g_ref[...] + beta_ref[...]

    ye = ln(he)
    yo = ln(ho)
    y = jnp.concatenate([ye[:, None, :], yo[:, None, :]], axis=1)
    o_ref[...] = y.reshape(2 * TBLK2, HID)


def _tc_transform_table(table, W, b, gamma, beta):
    v = table.shape[0]
    assert v % (2 * TBLK2) == 0
    table2 = table.reshape(v // 2, 2 * EMB)
    wc = W - jnp.mean(W, axis=1, keepdims=True)
    bc = (b - jnp.mean(b)).reshape(1, HID)
    zeros = jnp.zeros((EMB, HID), jnp.float32)
    w2 = jnp.concatenate(
        [jnp.concatenate([wc, zeros], axis=1),
         jnp.concatenate([zeros, wc], axis=1)], axis=0)
    b2 = jnp.tile(bc, (1, 2))
    grid = (v // (2 * TBLK2),)
    return pl.pallas_call(
        _dense_pairs_body,
        grid=grid,
        in_specs=[
            pl.BlockSpec((TBLK2, 2 * EMB), lambda i: (i, 0)),
            pl.BlockSpec((2 * EMB, 2 * HID), lambda i: (0, 0)),
            pl.BlockSpec((1, 2 * HID), lambda i: (0, 0)),
            pl.BlockSpec((1, HID), lambda i: (0, 0)),
            pl.BlockSpec((1, HID), lambda i: (0, 0)),
        ],
        out_specs=pl.BlockSpec((2 * TBLK2, HID), lambda i: (i, 0)),
        out_shape=jax.ShapeDtypeStruct((v, HID), jnp.float32),
        compiler_params=pltpu.CompilerParams(
            dimension_semantics=("arbitrary",)),
    )(table2, w2, b2, gamma.reshape(1, HID), beta.reshape(1, HID))


def _gather_body(f_hbm, idx_hbm, out_hbm, idx_v, rows_v, sem, *, b_per_w,
                 n_chunks):
    wid = lax.axis_index("s") * NC + lax.axis_index("c")
    base = wid * b_per_w

    def body(i, carry):
        off = base + i * CHUNK
        pltpu.sync_copy(idx_hbm.at[pl.ds(off, CHUNK)], idx_v)
        pltpu.async_copy(f_hbm.at[idx_v], rows_v, sem).wait()
        pltpu.sync_copy(rows_v, out_hbm.at[pl.ds(off, CHUNK)])
        return carry

    lax.fori_loop(0, n_chunks, body, 0)


def _sc_gather_rows(f, idx_flat):
    (b,) = idx_flat.shape
    assert b % (NW * CHUNK) == 0, b
    b_per_w = b // NW
    n_chunks = b_per_w // CHUNK
    mesh = plsc.VectorSubcoreMesh(core_axis_name="c", subcore_axis_name="s",
                                  num_cores=NC, num_subcores=NS)
    f_call = pl.kernel(
        functools.partial(_gather_body, b_per_w=b_per_w, n_chunks=n_chunks),
        out_type=jax.ShapeDtypeStruct((b, HID), jnp.float32),
        mesh=mesh,
        scratch_types=[
            pltpu.VMEM((CHUNK,), jnp.int32),
            pltpu.VMEM((CHUNK, HID), jnp.float32),
            pltpu.SemaphoreType.DMA,
        ],
    )
    return f_call(f, idx_flat)


def kernel(input_ids, table, W, b, gamma, beta):
    B, L = input_ids.shape
    ids_flat = input_ids.reshape(-1).astype(jnp.int32)
    f = _tc_transform_table(table, W, b, gamma, beta)
    out = _sc_gather_rows(f, ids_flat)
    return out.reshape(B, L, HID)


# speedup vs baseline: 1.1632x; 1.1632x over previous
import functools

import jax
import jax.numpy as jnp
from jax import lax
from jax.experimental import pallas as pl
from jax.experimental.pallas import tpu as pltpu
from jax.experimental.pallas import tpu_sc as plsc

EMB = 64
HID = 128
EPS = 1e-12

NC = 2
NS = 16
NW = NC * NS

CHUNK = 800

TBLK = 8000


def _dense_body(t_hbm, w_ref, b_ref, g_ref, beta_ref, o_ref, x_vmem, sem):
    i = pl.program_id(0)
    pltpu.async_copy(t_hbm.at[pl.ds(i * TBLK, TBLK), :], x_vmem, sem).wait()
    x = x_vmem[...]
    hc = jnp.dot(x, w_ref[...], preferred_element_type=jnp.float32)
    hc = hc + b_ref[...]
    var = jnp.mean(hc * hc, axis=-1, keepdims=True)
    o_ref[...] = hc * lax.rsqrt(var + EPS) * g_ref[...] + beta_ref[...]


def _tc_transform_table(table, W, b, gamma, beta):
    v = table.shape[0]
    assert v % TBLK == 0
    wc = W - jnp.mean(W, axis=1, keepdims=True)
    bc = (b - jnp.mean(b)).reshape(1, HID)
    grid = (v // TBLK,)
    return pl.pallas_call(
        _dense_body,
        grid=grid,
        in_specs=[
            pl.BlockSpec(memory_space=pl.ANY),
            pl.BlockSpec((EMB, HID), lambda i: (0, 0)),
            pl.BlockSpec((1, HID), lambda i: (0, 0)),
            pl.BlockSpec((1, HID), lambda i: (0, 0)),
            pl.BlockSpec((1, HID), lambda i: (0, 0)),
        ],
        out_specs=pl.BlockSpec((TBLK, HID), lambda i: (i, 0)),
        out_shape=jax.ShapeDtypeStruct((v, HID), jnp.float32),
        scratch_shapes=[
            pltpu.VMEM((TBLK, EMB), jnp.float32),
            pltpu.SemaphoreType.DMA,
        ],
        compiler_params=pltpu.CompilerParams(
            dimension_semantics=("arbitrary",)),
    )(table, wc, bc, gamma.reshape(1, HID), beta.reshape(1, HID))


def _gather_body(f_hbm, idx_hbm, out_hbm, idx_v, rows_v, sem, *, b_per_w,
                 n_chunks):
    wid = lax.axis_index("s") * NC + lax.axis_index("c")
    base = wid * b_per_w

    def body(i, carry):
        off = base + i * CHUNK
        pltpu.sync_copy(idx_hbm.at[pl.ds(off, CHUNK)], idx_v)
        pltpu.async_copy(f_hbm.at[idx_v], rows_v, sem).wait()
        pltpu.sync_copy(rows_v, out_hbm.at[pl.ds(off, CHUNK)])
        return carry

    lax.fori_loop(0, n_chunks, body, 0)


def _sc_gather_rows(f, idx_flat):
    (b,) = idx_flat.shape
    assert b % (NW * CHUNK) == 0, b
    b_per_w = b // NW
    n_chunks = b_per_w // CHUNK
    mesh = plsc.VectorSubcoreMesh(core_axis_name="c", subcore_axis_name="s",
                                  num_cores=NC, num_subcores=NS)
    f_call = pl.kernel(
        functools.partial(_gather_body, b_per_w=b_per_w, n_chunks=n_chunks),
        out_type=jax.ShapeDtypeStruct((b, HID), jnp.float32),
        mesh=mesh,
        scratch_types=[
            pltpu.VMEM((CHUNK,), jnp.int32),
            pltpu.VMEM((CHUNK, HID), jnp.float32),
            pltpu.SemaphoreType.DMA,
        ],
    )
    return f_call(f, idx_flat)


def kernel(input_ids, table, W, b, gamma, beta):
    B, L = input_ids.shape
    ids_flat = input_ids.reshape(-1).astype(jnp.int32)
    f = _tc_transform_table(table, W, b, gamma, beta)
    out = _sc_gather_rows(f, ids_flat)
    return out.reshape(B, L, HID)


# speedup vs baseline: 1.5777x; 1.3564x over previous
import functools

import jax
import jax.numpy as jnp
from jax import lax
from jax.experimental import pallas as pl
from jax.experimental.pallas import tpu as pltpu
from jax.experimental.pallas import tpu_sc as plsc

EMB = 64
HID = 128
EPS = 1e-12

NC = 2
NS = 16
NW = NC * NS

CHUNK = 800

TBLK = 8000


def _dense_body(t_ref, w_ref, b_ref, g_ref, beta_ref, o_ref):
    x = t_ref[...]
    hc = jnp.dot(x, w_ref[...], preferred_element_type=jnp.float32)
    hc = hc + b_ref[...]
    var = jnp.mean(hc * hc, axis=-1, keepdims=True)
    o_ref[...] = hc * lax.rsqrt(var + EPS) * g_ref[...] + beta_ref[...]


def _tc_transform_table(table, W, b, gamma, beta):
    v = table.shape[0]
    assert v % TBLK == 0
    tb = table.astype(jnp.bfloat16)
    wc = (W - jnp.mean(W, axis=1, keepdims=True)).astype(jnp.bfloat16)
    bc = (b - jnp.mean(b)).reshape(1, HID)
    grid = (v // TBLK,)
    return pl.pallas_call(
        _dense_body,
        grid=grid,
        in_specs=[
            pl.BlockSpec((TBLK, EMB), lambda i: (i, 0)),
            pl.BlockSpec((EMB, HID), lambda i: (0, 0)),
            pl.BlockSpec((1, HID), lambda i: (0, 0)),
            pl.BlockSpec((1, HID), lambda i: (0, 0)),
            pl.BlockSpec((1, HID), lambda i: (0, 0)),
        ],
        out_specs=pl.BlockSpec((TBLK, HID), lambda i: (i, 0)),
        out_shape=jax.ShapeDtypeStruct((v, HID), jnp.float32),
        compiler_params=pltpu.CompilerParams(
            dimension_semantics=("arbitrary",)),
    )(tb, wc, bc, gamma.reshape(1, HID), beta.reshape(1, HID))


def _gather_body(f_hbm, idx_hbm, out_hbm, idx_v, rows_v, sem, *, b_per_w,
                 n_chunks):
    wid = lax.axis_index("s") * NC + lax.axis_index("c")
    base = wid * b_per_w

    def body(i, carry):
        off = base + i * CHUNK
        pltpu.sync_copy(idx_hbm.at[pl.ds(off, CHUNK)], idx_v)
        pltpu.async_copy(f_hbm.at[idx_v], rows_v, sem).wait()
        pltpu.sync_copy(rows_v, out_hbm.at[pl.ds(off, CHUNK)])
        return carry

    lax.fori_loop(0, n_chunks, body, 0)


def _sc_gather_rows(f, idx_flat):
    (b,) = idx_flat.shape
    assert b % (NW * CHUNK) == 0, b
    b_per_w = b // NW
    n_chunks = b_per_w // CHUNK
    mesh = plsc.VectorSubcoreMesh(core_axis_name="c", subcore_axis_name="s",
                                  num_cores=NC, num_subcores=NS)
    f_call = pl.kernel(
        functools.partial(_gather_body, b_per_w=b_per_w, n_chunks=n_chunks),
        out_type=jax.ShapeDtypeStruct((b, HID), jnp.float32),
        mesh=mesh,
        scratch_types=[
            pltpu.VMEM((CHUNK,), jnp.int32),
            pltpu.VMEM((CHUNK, HID), jnp.float32),
            pltpu.SemaphoreType.DMA,
        ],
    )
    return f_call(f, idx_flat)


def kernel(input_ids, table, W, b, gamma, beta):
    B, L = input_ids.shape
    ids_flat = input_ids.reshape(-1).astype(jnp.int32)
    f = _tc_transform_table(table, W, b, gamma, beta)
    out = _sc_gather_rows(f, ids_flat)
    return out.reshape(B, L, HID)


# speedup vs baseline: 1.6058x; 1.0178x over previous
import functools

import jax
import jax.numpy as jnp
from jax import lax
from jax.experimental import pallas as pl
from jax.experimental.pallas import tpu as pltpu
from jax.experimental.pallas import tpu_sc as plsc

EMB = 64
HID = 128
EPS = 1e-12

NC = 2
NS = 16
NW = NC * NS

CHUNK = 400

TBLK = 8000


def _dense_body(t_ref, w_ref, b_ref, g_ref, beta_ref, o_ref):
    x = t_ref[...]
    hc = jnp.dot(x, w_ref[...], preferred_element_type=jnp.float32)
    hc = hc + b_ref[...]
    var = jnp.mean(hc * hc, axis=-1, keepdims=True)
    o_ref[...] = hc * lax.rsqrt(var + EPS) * g_ref[...] + beta_ref[...]


def _tc_transform_table(table, W, b, gamma, beta):
    v = table.shape[0]
    assert v % TBLK == 0
    tb = table.astype(jnp.bfloat16)
    wc = (W - jnp.mean(W, axis=1, keepdims=True)).astype(jnp.bfloat16)
    bc = (b - jnp.mean(b)).reshape(1, HID)
    grid = (v // TBLK,)
    return pl.pallas_call(
        _dense_body,
        grid=grid,
        in_specs=[
            pl.BlockSpec((TBLK, EMB), lambda i: (i, 0)),
            pl.BlockSpec((EMB, HID), lambda i: (0, 0)),
            pl.BlockSpec((1, HID), lambda i: (0, 0)),
            pl.BlockSpec((1, HID), lambda i: (0, 0)),
            pl.BlockSpec((1, HID), lambda i: (0, 0)),
        ],
        out_specs=pl.BlockSpec((TBLK, HID), lambda i: (i, 0)),
        out_shape=jax.ShapeDtypeStruct((v, HID), jnp.float32),
        compiler_params=pltpu.CompilerParams(
            dimension_semantics=("arbitrary",)),
    )(tb, wc, bc, gamma.reshape(1, HID), beta.reshape(1, HID))


def _gather_body(f_hbm, idx_hbm, out_hbm, idx_v0, idx_v1, rows_v0, rows_v1,
                 sem0, sem1, *, b_per_w, n_chunks):
    wid = lax.axis_index("s") * NC + lax.axis_index("c")
    base = wid * b_per_w
    n_pairs = n_chunks // 2

    def start(c, idx_v, rows_v, sem):
        pltpu.sync_copy(idx_hbm.at[pl.ds(base + c * CHUNK, CHUNK)], idx_v)
        pltpu.async_copy(f_hbm.at[idx_v], rows_v, sem)

    def drain(c, idx_v, rows_v, sem):
        pltpu.make_async_copy(f_hbm.at[idx_v], rows_v, sem).wait()
        pltpu.sync_copy(rows_v, out_hbm.at[pl.ds(base + c * CHUNK, CHUNK)])

    start(0, idx_v0, rows_v0, sem0)

    def body(j, carry):
        a = 2 * j
        start(a + 1, idx_v1, rows_v1, sem1)
        drain(a, idx_v0, rows_v0, sem0)

        @pl.when(j < n_pairs - 1)
        def _():
            start(a + 2, idx_v0, rows_v0, sem0)

        drain(a + 1, idx_v1, rows_v1, sem1)
        return carry

    lax.fori_loop(0, n_pairs, body, 0)


def _sc_gather_rows(f, idx_flat):
    (b,) = idx_flat.shape
    assert b % (NW * CHUNK) == 0, b
    b_per_w = b // NW
    n_chunks = b_per_w // CHUNK
    mesh = plsc.VectorSubcoreMesh(core_axis_name="c", subcore_axis_name="s",
                                  num_cores=NC, num_subcores=NS)
    f_call = pl.kernel(
        functools.partial(_gather_body, b_per_w=b_per_w, n_chunks=n_chunks),
        out_type=jax.ShapeDtypeStruct((b, HID), jnp.float32),
        mesh=mesh,
        scratch_types=[
            pltpu.VMEM((CHUNK,), jnp.int32),
            pltpu.VMEM((CHUNK,), jnp.int32),
            pltpu.VMEM((CHUNK, HID), jnp.float32),
            pltpu.VMEM((CHUNK, HID), jnp.float32),
            pltpu.SemaphoreType.DMA,
            pltpu.SemaphoreType.DMA,
        ],
    )
    return f_call(f, idx_flat)


def kernel(input_ids, table, W, b, gamma, beta):
    B, L = input_ids.shape
    ids_flat = input_ids.reshape(-1).astype(jnp.int32)
    f = _tc_transform_table(table, W, b, gamma, beta)
    out = _sc_gather_rows(f, ids_flat)
    return out.reshape(B, L, HID)


# speedup vs baseline: 1.6644x; 1.0365x over previous
import functools

import jax
import jax.numpy as jnp
from jax import lax
from jax.experimental import pallas as pl
from jax.experimental.pallas import tpu as pltpu
from jax.experimental.pallas import tpu_sc as plsc

EMB = 64
HID = 128
EPS = 1e-12

NC = 2
NS = 16
NW = NC * NS

CHUNK = 400

TBLK = 20000


def _dense_body(t_ref, w_ref, b_ref, g_ref, beta_ref, o_ref):
    x = t_ref[...]
    hc = jnp.dot(x, w_ref[...], preferred_element_type=jnp.float32)
    hc = hc + b_ref[...]
    var = jnp.mean(hc * hc, axis=-1, keepdims=True)
    o_ref[...] = hc * lax.rsqrt(var + EPS) * g_ref[...] + beta_ref[...]


def _tc_transform_table(table, W, b, gamma, beta):
    v = table.shape[0]
    assert v % TBLK == 0
    tb = table.astype(jnp.bfloat16)
    wc = (W - jnp.mean(W, axis=1, keepdims=True)).astype(jnp.bfloat16)
    bc = (b - jnp.mean(b)).reshape(1, HID)
    grid = (v // TBLK,)
    return pl.pallas_call(
        _dense_body,
        grid=grid,
        in_specs=[
            pl.BlockSpec((TBLK, EMB), lambda i: (i, 0)),
            pl.BlockSpec((EMB, HID), lambda i: (0, 0)),
            pl.BlockSpec((1, HID), lambda i: (0, 0)),
            pl.BlockSpec((1, HID), lambda i: (0, 0)),
            pl.BlockSpec((1, HID), lambda i: (0, 0)),
        ],
        out_specs=pl.BlockSpec((TBLK, HID), lambda i: (i, 0)),
        out_shape=jax.ShapeDtypeStruct((v, HID), jnp.float32),
        compiler_params=pltpu.CompilerParams(
            dimension_semantics=("arbitrary",)),
    )(tb, wc, bc, gamma.reshape(1, HID), beta.reshape(1, HID))


def _gather_body(f_hbm, idx_hbm, out_hbm, idx_v0, idx_v1, rows_v0, rows_v1,
                 sem0, sem1, *, b_per_w, n_chunks):
    wid = lax.axis_index("s") * NC + lax.axis_index("c")
    base = wid * b_per_w
    n_pairs = n_chunks // 2

    def start(c, idx_v, rows_v, sem):
        pltpu.sync_copy(idx_hbm.at[pl.ds(base + c * CHUNK, CHUNK)], idx_v)
        pltpu.async_copy(f_hbm.at[idx_v], rows_v, sem)

    def drain(c, idx_v, rows_v, sem):
        pltpu.make_async_copy(f_hbm.at[idx_v], rows_v, sem).wait()
        pltpu.sync_copy(rows_v, out_hbm.at[pl.ds(base + c * CHUNK, CHUNK)])

    start(0, idx_v0, rows_v0, sem0)

    def body(j, carry):
        a = 2 * j
        start(a + 1, idx_v1, rows_v1, sem1)
        drain(a, idx_v0, rows_v0, sem0)

        @pl.when(j < n_pairs - 1)
        def _():
            start(a + 2, idx_v0, rows_v0, sem0)

        drain(a + 1, idx_v1, rows_v1, sem1)
        return carry

    lax.fori_loop(0, n_pairs, body, 0)


def _sc_gather_rows(f, idx_flat):
    (b,) = idx_flat.shape
    assert b % (NW * CHUNK) == 0, b
    b_per_w = b // NW
    n_chunks = b_per_w // CHUNK
    mesh = plsc.VectorSubcoreMesh(core_axis_name="c", subcore_axis_name="s",
                                  num_cores=NC, num_subcores=NS)
    f_call = pl.kernel(
        functools.partial(_gather_body, b_per_w=b_per_w, n_chunks=n_chunks),
        out_type=jax.ShapeDtypeStruct((b, HID), jnp.float32),
        mesh=mesh,
        scratch_types=[
            pltpu.VMEM((CHUNK,), jnp.int32),
            pltpu.VMEM((CHUNK,), jnp.int32),
            pltpu.VMEM((CHUNK, HID), jnp.float32),
            pltpu.VMEM((CHUNK, HID), jnp.float32),
            pltpu.SemaphoreType.DMA,
            pltpu.SemaphoreType.DMA,
        ],
    )
    return f_call(f, idx_flat)


def kernel(input_ids, table, W, b, gamma, beta):
    B, L = input_ids.shape
    ids_flat = input_ids.reshape(-1).astype(jnp.int32)
    f = _tc_transform_table(table, W, b, gamma, beta)
    out = _sc_gather_rows(f, ids_flat)
    return out.reshape(B, L, HID)


# speedup vs baseline: 1.6719x; 1.0045x over previous
import functools

import jax
import jax.numpy as jnp
from jax import lax
from jax.experimental import pallas as pl
from jax.experimental.pallas import tpu as pltpu
from jax.experimental.pallas import tpu_sc as plsc

EMB = 64
HID = 128
EPS = 1e-12

NC = 2
NS = 16
NW = NC * NS

CHUNK = 400

TBLK = 25000


def _dense_body(t_ref, w_ref, b_ref, g_ref, beta_ref, o_ref):
    x = t_ref[...]
    hc = jnp.dot(x, w_ref[...], preferred_element_type=jnp.float32)
    hc = hc + b_ref[...]
    var = jnp.mean(hc * hc, axis=-1, keepdims=True)
    o_ref[...] = hc * lax.rsqrt(var + EPS) * g_ref[...] + beta_ref[...]


def _tc_transform_table(table, W, b, gamma, beta):
    v = table.shape[0]
    assert v % TBLK == 0
    tb = table.astype(jnp.bfloat16)
    wc = (W - jnp.mean(W, axis=1, keepdims=True)).astype(jnp.bfloat16)
    bc = (b - jnp.mean(b)).reshape(1, HID)
    grid = (v // TBLK,)
    return pl.pallas_call(
        _dense_body,
        grid=grid,
        in_specs=[
            pl.BlockSpec((TBLK, EMB), lambda i: (i, 0)),
            pl.BlockSpec((EMB, HID), lambda i: (0, 0)),
            pl.BlockSpec((1, HID), lambda i: (0, 0)),
            pl.BlockSpec((1, HID), lambda i: (0, 0)),
            pl.BlockSpec((1, HID), lambda i: (0, 0)),
        ],
        out_specs=pl.BlockSpec((TBLK, HID), lambda i: (i, 0)),
        out_shape=jax.ShapeDtypeStruct((v, HID), jnp.float32),
        compiler_params=pltpu.CompilerParams(
            dimension_semantics=("arbitrary",)),
    )(tb, wc, bc, gamma.reshape(1, HID), beta.reshape(1, HID))


def _gather_body(f_hbm, idx_hbm, out_hbm, idx_v0, idx_v1, rows_v0, rows_v1,
                 sem0, sem1, *, b_per_w, n_chunks):
    wid = lax.axis_index("s") * NC + lax.axis_index("c")
    base = wid * b_per_w
    n_pairs = n_chunks // 2

    def start(c, idx_v, rows_v, sem):
        pltpu.sync_copy(idx_hbm.at[pl.ds(base + c * CHUNK, CHUNK)], idx_v)
        pltpu.async_copy(f_hbm.at[idx_v], rows_v, sem)

    def drain(c, idx_v, rows_v, sem):
        pltpu.make_async_copy(f_hbm.at[idx_v], rows_v, sem).wait()
        pltpu.sync_copy(rows_v, out_hbm.at[pl.ds(base + c * CHUNK, CHUNK)])

    start(0, idx_v0, rows_v0, sem0)

    def body(j, carry):
        a = 2 * j
        start(a + 1, idx_v1, rows_v1, sem1)
        drain(a, idx_v0, rows_v0, sem0)

        @pl.when(j < n_pairs - 1)
        def _():
            start(a + 2, idx_v0, rows_v0, sem0)

        drain(a + 1, idx_v1, rows_v1, sem1)
        return carry

    lax.fori_loop(0, n_pairs, body, 0)


def _sc_gather_rows(f, idx_flat):
    (b,) = idx_flat.shape
    assert b % (NW * CHUNK) == 0, b
    b_per_w = b // NW
    n_chunks = b_per_w // CHUNK
    mesh = plsc.VectorSubcoreMesh(core_axis_name="c", subcore_axis_name="s",
                                  num_cores=NC, num_subcores=NS)
    f_call = pl.kernel(
        functools.partial(_gather_body, b_per_w=b_per_w, n_chunks=n_chunks),
        out_type=jax.ShapeDtypeStruct((b, HID), jnp.float32),
        mesh=mesh,
        scratch_types=[
            pltpu.VMEM((CHUNK,), jnp.int32),
            pltpu.VMEM((CHUNK,), jnp.int32),
            pltpu.VMEM((CHUNK, HID), jnp.float32),
            pltpu.VMEM((CHUNK, HID), jnp.float32),
            pltpu.SemaphoreType.DMA,
            pltpu.SemaphoreType.DMA,
        ],
    )
    return f_call(f, idx_flat)


def kernel(input_ids, table, W, b, gamma, beta):
    B, L = input_ids.shape
    ids_flat = input_ids.reshape(-1).astype(jnp.int32)
    f = _tc_transform_table(table, W, b, gamma, beta)
    out = _sc_gather_rows(f, ids_flat)
    return out.reshape(B, L, HID)
